# baseline (speedup 1.0000x reference)
import functools

import jax
import jax.numpy as jnp
from jax import lax
from jax.experimental import pallas as pl
from jax.experimental.pallas import tpu as pltpu
from jax.experimental.pallas import tpu_sc as plsc

NUM_HIDDENS = 1024
B_TOTAL = 4 * 4096
NC = 2
NS = 16
NW = NC * NS
B_PER_W = B_TOTAL // NW
CHUNK = 32
NCHUNK = B_PER_W // CHUNK
NBUF = 3


def _make_gather():
    mesh = plsc.VectorSubcoreMesh(core_axis_name="c", subcore_axis_name="s")

    @functools.partial(
        pl.kernel,
        mesh=mesh,
        out_type=jax.ShapeDtypeStruct((B_TOTAL, NUM_HIDDENS), jnp.float32),
        scratch_types=[
            pltpu.VMEM((B_PER_W,), jnp.int32),
            pltpu.VMEM((NBUF, CHUNK, NUM_HIDDENS), jnp.float32),
        ]
        + [pltpu.SemaphoreType.DMA] * (2 * NBUF),
    )
    def k(table_hbm, idx_hbm, out_hbm, idx_v, rows_v, *sems):
        wid = lax.axis_index("s") * NC + lax.axis_index("c")
        base = wid * B_PER_W
        gsem = sems[:NBUF]
        ssem = sems[NBUF:]
        pltpu.sync_copy(idx_hbm.at[pl.ds(base, B_PER_W)], idx_v)

        def start_gather(g, b):
            return pltpu.async_copy(
                table_hbm.at[idx_v.at[pl.ds(g * CHUNK, CHUNK)]],
                rows_v.at[b],
                gsem[b],
            )

        def start_scatter(g, b):
            return pltpu.async_copy(
                rows_v.at[b],
                out_hbm.at[pl.ds(base + g * CHUNK, CHUNK)],
                ssem[b],
            )

        gather_h = [None] * NBUF
        scatter_h = [None] * NBUF
        gather_h[0] = start_gather(0, 0)
        for g in range(NCHUNK):
            b = g % NBUF
            if g + 1 < NCHUNK:
                nb = (g + 1) % NBUF
                if scatter_h[nb] is not None:
                    scatter_h[nb].wait()
                gather_h[nb] = start_gather(g + 1, nb)
            gather_h[b].wait()
            scatter_h[b] = start_scatter(g, b)
        for b in range(NBUF):
            if scatter_h[b] is not None:
                scatter_h[b].wait()

    return k


_gather = _make_gather()


def kernel(position_ids, PosEnc):
    ids = position_ids.reshape(-1).astype(jnp.int32)
    out = _gather(PosEnc, ids)
    return out.reshape(position_ids.shape + (NUM_HIDDENS,))


# speedup vs baseline: 1.0004x; 1.0004x over previous
import functools

import jax
import jax.numpy as jnp
from jax import lax
from jax.experimental import pallas as pl
from jax.experimental.pallas import tpu as pltpu
from jax.experimental.pallas import tpu_sc as plsc

NUM_HIDDENS = 1024
B_TOTAL = 4 * 4096
NC = 2
NS = 16
NW = NC * NS
B_PER_W = B_TOTAL // NW
CHUNK = 16
NCHUNK = B_PER_W // CHUNK
NBUF = 6


def _make_gather():
    mesh = plsc.VectorSubcoreMesh(core_axis_name="c", subcore_axis_name="s")

    @functools.partial(
        pl.kernel,
        mesh=mesh,
        out_type=jax.ShapeDtypeStruct((B_TOTAL, NUM_HIDDENS), jnp.float32),
        scratch_types=[
            pltpu.VMEM((B_PER_W,), jnp.int32),
            pltpu.VMEM((NBUF, CHUNK, NUM_HIDDENS), jnp.float32),
        ]
        + [pltpu.SemaphoreType.DMA] * (2 * NBUF),
    )
    def k(table_hbm, idx_hbm, out_hbm, idx_v, rows_v, *sems):
        wid = lax.axis_index("s") * NC + lax.axis_index("c")
        base = wid * B_PER_W
        gsem = sems[:NBUF]
        ssem = sems[NBUF:]
        pltpu.sync_copy(idx_hbm.at[pl.ds(base, B_PER_W)], idx_v)

        def start_gather(g, b):
            return pltpu.async_copy(
                table_hbm.at[idx_v.at[pl.ds(g * CHUNK, CHUNK)]],
                rows_v.at[b],
                gsem[b],
            )

        def start_scatter(g, b):
            return pltpu.async_copy(
                rows_v.at[b],
                out_hbm.at[pl.ds(base + g * CHUNK, CHUNK)],
                ssem[b],
            )

        gather_h = [None] * NBUF
        scatter_h = [None] * NBUF
        gather_h[0] = start_gather(0, 0)
        for g in range(NCHUNK):
            b = g % NBUF
            if g + 1 < NCHUNK:
                nb = (g + 1) % NBUF
                if scatter_h[nb] is not None:
                    scatter_h[nb].wait()
                gather_h[nb] = start_gather(g + 1, nb)
            gather_h[b].wait()
            scatter_h[b] = start_scatter(g, b)
        for b in range(NBUF):
            if scatter_h[b] is not None:
                scatter_h[b].wait()

    return k


_gather = _make_gather()


def kernel(position_ids, PosEnc):
    ids = position_ids.reshape(-1).astype(jnp.int32)
    out = _gather(PosEnc, ids)
    return out.reshape(position_ids.shape + (NUM_HIDDENS,))
